# baseline (speedup 1.0000x reference)
import functools

import jax
import jax.numpy as jnp
from jax import lax
from jax.experimental import pallas as pl
from jax.experimental.pallas import tpu as pltpu
from jax.experimental.pallas import tpu_sc as plsc

N = 16384
D = 32
TYPE_ROWS = 1000
BIG_ROWS = 100000
_NC = 2
_NS = 16
_NW = _NC * _NS
_BPW = N // _NW

_C = 2048


def _eye_off(k):
    r = lax.broadcasted_iota(jnp.int32, (32, 128), 0)
    c = lax.broadcasted_iota(jnp.int32, (32, 128), 1)
    return jnp.where(c == r + 32 * k, 1.0, 0.0).astype(jnp.float32)


def _mxu_t4(xs):
    acc = None
    for k, x in enumerate(xs):
        p = lax.dot_general(x, _eye_off(k), (((0,), (0,)), ((), ())),
                            preferred_element_type=jnp.float32)
        acc = p if acc is None else acc + p
    return acc


def _relayout_body(p0, p1, p2, p3, op):
    i = pl.program_id(0)
    col = lax.broadcasted_iota(jnp.int32, (32, _C), 1)
    ps = [p0, p1, p2, p3]
    ok = [(4 * i + k) * _C + col < BIG_ROWS for k in range(4)]
    op[...] = _mxu_t4([jnp.where(ok[k], ps[k][...], 0.0) for k in range(4)])


def _tc_relayout_big(table_t):
    v = table_t.shape[1]
    g = -(-v // _C)
    nsteps = -(-g // 4)

    def spec(k):
        return pl.BlockSpec(
            (32, _C), lambda i, k=k: (0, jnp.minimum(4 * i + k, g - 1)))

    return pl.pallas_call(
        _relayout_body,
        grid=(nsteps,),
        in_specs=[spec(0), spec(1), spec(2), spec(3)],
        out_specs=pl.BlockSpec((_C, 128), lambda i: (i, 0)),
        out_shape=jax.ShapeDtypeStruct((nsteps * _C, 128), jnp.float32),
        compiler_params=pltpu.CompilerParams(
            dimension_semantics=("parallel",)),
    )(table_t, table_t, table_t, table_t)


def _perm_big(idx):
    b = idx // _C
    return ((b // 4) * _C + idx % _C) * 4 + b % 4


def _relayout_type_body(tin, o):
    q = TYPE_ROWS // 4
    for k in range(4):
        o[:, 32 * k:32 * k + 32] = jnp.transpose(tin[:, k * q:(k + 1) * q])


def _tc_relayout_type(table_t):
    q = TYPE_ROWS // 4
    return pl.pallas_call(
        _relayout_type_body,
        grid=(1,),
        in_specs=[pl.BlockSpec((32, TYPE_ROWS), lambda i: (0, 0))],
        out_specs=pl.BlockSpec((q, 128), lambda i: (0, 0)),
        out_shape=jax.ShapeDtypeStruct((q, 128), jnp.float32),
    )(table_t)


def _sc_gather2(table_a, table_b, idx_a, idx_b):
    mesh = plsc.VectorSubcoreMesh(core_axis_name="c", subcore_axis_name="s")

    @functools.partial(
        pl.kernel,
        mesh=mesh,
        out_type=[jax.ShapeDtypeStruct((N, D), jnp.float32)] * 2,
        scratch_types=[
            pltpu.VMEM((_BPW,), jnp.int32),
            pltpu.VMEM((_BPW,), jnp.int32),
            pltpu.VMEM((_BPW, D), jnp.float32),
            pltpu.VMEM((_BPW, D), jnp.float32),
            pltpu.SemaphoreType.DMA,
            pltpu.SemaphoreType.DMA,
        ],
        compiler_params=pltpu.CompilerParams(use_tc_tiling_on_sc=False),
    )
    def k(ta, tb, ia_, ib_, o_a, o_b, iv_a, iv_b, r_a, r_b, s_a, s_b):
        wid = lax.axis_index("s") * _NC + lax.axis_index("c")
        base = wid * _BPW
        pltpu.sync_copy(ia_.at[pl.ds(base, _BPW)], iv_a)
        pltpu.sync_copy(ib_.at[pl.ds(base, _BPW)], iv_b)
        c1 = pltpu.async_copy(ta.at[iv_a], r_a, s_a)
        c2 = pltpu.async_copy(tb.at[iv_b], r_b, s_b)
        c1.wait()
        c2.wait()
        pltpu.sync_copy(r_a, o_a.at[pl.ds(base, _BPW)])
        pltpu.sync_copy(r_b, o_b.at[pl.ds(base, _BPW)])

    return k(table_a, table_b, idx_a, idx_b)


def _sc_gather1(table, idx):
    mesh = plsc.VectorSubcoreMesh(core_axis_name="c", subcore_axis_name="s")

    @functools.partial(
        pl.kernel,
        mesh=mesh,
        out_type=jax.ShapeDtypeStruct((N, D), jnp.float32),
        scratch_types=[
            pltpu.VMEM((_BPW,), jnp.int32),
            pltpu.VMEM((_BPW, D), jnp.float32),
            pltpu.SemaphoreType.DMA,
        ],
        compiler_params=pltpu.CompilerParams(use_tc_tiling_on_sc=False),
    )
    def k(ta, ia_, o_a, iv_a, r_a, s_a):
        wid = lax.axis_index("s") * _NC + lax.axis_index("c")
        base = wid * _BPW
        pltpu.sync_copy(ia_.at[pl.ds(base, _BPW)], iv_a)
        pltpu.async_copy(ta.at[iv_a], r_a, s_a).wait()
        pltpu.sync_copy(r_a, o_a.at[pl.ds(base, _BPW)])

    return k(table, idx)


def _gelu(x):
    return x * 0.5 * (1.0 + lax.erf(x * 0.7071067811865476))


def _ln(x, g, b):
    m = jnp.mean(x, axis=-1, keepdims=True)
    d = x - m
    v = jnp.mean(d * d, axis=-1, keepdims=True)
    return d * lax.rsqrt(v + 1e-5) * g + b


def _avg_mat(w):
    return jnp.full((w, w), 1.0 / w, dtype=jnp.float32)


def _ln_mxu(x, g, b):
    mm = _avg_mat(x.shape[-1])
    m = jnp.dot(x, mm, preferred_element_type=jnp.float32)
    d = x - m
    v = jnp.dot(d * d, mm, preferred_element_type=jnp.float32)
    return d * lax.rsqrt(v + 1e-5) * g + b


def _dense_body(te, pe, me, db,
                dbW, dbb, dbg, dbbeta,
                tmW, tmb, tmg, tmbeta,
                tfW, tfb, tfg, tfbeta,
                fW1, fW2, fW3, fW4, fb, fg, fbeta, o):
    f32 = jnp.float32
    b4 = te.shape[0]
    tec = jnp.concatenate([te[:, 32 * a:32 * a + 32] for a in range(4)], 0)
    pec = jnp.concatenate([pe[:, 32 * a:32 * a + 32] for a in range(4)], 0)
    mec = jnp.concatenate([me[:, 32 * a:32 * a + 32] for a in range(4)], 0)
    dbc = jnp.concatenate([db[:, 2 * a:2 * a + 2] for a in range(4)], 0)
    db_h = jnp.dot(dbc, dbW[...], preferred_element_type=f32) + dbb[...]
    db_e = _gelu(_ln_mxu(db_h, dbg[...], dbbeta[...]))
    t = jnp.dot(mec, tmW[...], preferred_element_type=f32) + tmb[...]
    t = _gelu(_ln_mxu(t, tmg[...], tmbeta[...]))
    t = jnp.dot(t, tfW[...], preferred_element_type=f32) + tfb[...]
    t = _gelu(_ln_mxu(t, tfg[...], tfbeta[...]))
    acc = (jnp.dot(tec, fW1[...], preferred_element_type=f32)
           + jnp.dot(pec, fW2[...], preferred_element_type=f32)
           + jnp.dot(db_e, fW3[...], preferred_element_type=f32)
           + jnp.dot(t, fW4[...], preferred_element_type=f32)
           + fb[...])
    res = _gelu(_ln_mxu(acc, fg[...], fbeta[...]))
    packed = jnp.stack([res[a * b4:(a + 1) * b4] for a in range(4)], 1)
    o[...] = packed.reshape(o.shape)


def _tc_dense(te4, pe4, me4, db4,
              db_W, db_b, db_g, db_beta,
              tm_W, tm_b, tm_g, tm_beta,
              tfin_W, tfin_b, tfin_g, tfin_beta,
              fin_W, fin_b, fin_g, fin_beta,
              block=4096):
    grid = (N // block,)
    b4 = block // 4

    def row(d):
        return pl.BlockSpec((b4, d), lambda i: (i, 0))

    def full(a):
        return pl.BlockSpec(a.shape, lambda i: (0,) * a.ndim)

    fW1 = fin_W[0:32]
    fW2 = fin_W[32:64]
    fW3 = fin_W[64:96]
    fW4 = fin_W[96:160]
    vecs = [db_b, db_g, db_beta, tm_b, tm_g, tm_beta,
            tfin_b, tfin_g, tfin_beta, fin_b, fin_g, fin_beta]
    (db_b, db_g, db_beta, tm_b, tm_g, tm_beta,
     tfin_b, tfin_g, tfin_beta, fin_b, fin_g, fin_beta) = [
        v.reshape(1, -1) for v in vecs]

    args = (te4, pe4, me4, db4,
            db_W, db_b, db_g, db_beta,
            tm_W, tm_b, tm_g, tm_beta,
            tfin_W, tfin_b, tfin_g, tfin_beta,
            fW1, fW2, fW3, fW4, fin_b, fin_g, fin_beta)
    specs = [row(128), row(128), row(128), row(8)] + [full(a) for a in args[4:]]

    return pl.pallas_call(
        _dense_body,
        grid=grid,
        in_specs=specs,
        out_specs=pl.BlockSpec((block, 128), lambda i: (i, 0)),
        out_shape=jax.ShapeDtypeStruct((N, 128), jnp.float32),
        compiler_params=pltpu.CompilerParams(
            dimension_semantics=("parallel",)),
    )(*args)


def kernel(features, type_table, pos_table, table_meta,
           db_W, db_b, db_g, db_beta,
           tm_W, tm_b, tm_g, tm_beta,
           tfin_W, tfin_b, tfin_g, tfin_beta,
           fin_W, fin_b, fin_g, fin_beta):
    idx_type = features[:, 0].astype(jnp.int32)
    idx_pos = features[:, 1].astype(jnp.int32)
    idx_tab = features[:, 6].astype(jnp.int32)
    db4 = (features.T[2:4].reshape(2, N // 4, 4)
           .transpose(1, 2, 0).reshape(N // 4, 8))
    tt_lin = _tc_relayout_type(type_table.T).reshape(TYPE_ROWS, D)
    pt_lin = _tc_relayout_big(pos_table.T)
    pt_lin = pt_lin.reshape(pt_lin.shape[0] * 4, D)
    idx_type = (idx_type % (TYPE_ROWS // 4)) * 4 + idx_type // (TYPE_ROWS // 4)
    idx_pos = _perm_big(idx_pos)
    idx_tab = _perm_big(idx_tab)
    te, pe = _sc_gather2(tt_lin, pt_lin, idx_type, idx_pos)
    tm_lin = _tc_relayout_big(table_meta.T)
    tm_lin = tm_lin.reshape(tm_lin.shape[0] * 4, D)
    me = _sc_gather1(tm_lin, idx_tab)
    te4 = te.reshape(N // 4, 128)
    pe4 = pe.reshape(N // 4, 128)
    me4 = me.reshape(N // 4, 128)
    return _tc_dense(te4, pe4, me4, db4,
                     db_W, db_b, db_g, db_beta,
                     tm_W, tm_b, tm_g, tm_beta,
                     tfin_W, tfin_b, tfin_g, tfin_beta,
                     fin_W, fin_b, fin_g, fin_beta)


# speedup vs baseline: 1.0961x; 1.0961x over previous
import functools

import jax
import jax.numpy as jnp
from jax import lax
from jax.experimental import pallas as pl
from jax.experimental.pallas import tpu as pltpu
from jax.experimental.pallas import tpu_sc as plsc

N = 16384
D = 32
TYPE_ROWS = 1000
BIG_ROWS = 100000
_NC = 2
_NS = 16
_NW = _NC * _NS
_BPW = N // _NW

_C = 2048


def _eye_off(k):
    r = lax.broadcasted_iota(jnp.int32, (32, 128), 0)
    c = lax.broadcasted_iota(jnp.int32, (32, 128), 1)
    return jnp.where(c == r + 32 * k, 1.0, 0.0).astype(jnp.float32)


def _mxu_t4(xs):
    acc = None
    for k, x in enumerate(xs):
        p = lax.dot_general(x, _eye_off(k), (((0,), (0,)), ((), ())),
                            preferred_element_type=jnp.float32)
        acc = p if acc is None else acc + p
    return acc


def _relayout_body(p0, p1, p2, p3, m0, m1, m2, m3, op, om):
    i = pl.program_id(0)
    col = lax.broadcasted_iota(jnp.int32, (32, _C), 1)
    ps, ms = [p0, p1, p2, p3], [m0, m1, m2, m3]
    ok = [(4 * i + k) * _C + col < BIG_ROWS for k in range(4)]
    op[...] = _mxu_t4([jnp.where(ok[k], ps[k][...], 0.0) for k in range(4)])
    om[...] = _mxu_t4([jnp.where(ok[k], ms[k][...], 0.0) for k in range(4)])


def _tc_relayout_big(pos_t, meta_t):
    v = pos_t.shape[1]
    g = -(-v // _C)
    nsteps = -(-g // 4)

    def spec(k):
        return pl.BlockSpec(
            (32, _C), lambda i, k=k: (0, jnp.minimum(4 * i + k, g - 1)))

    specs = [spec(0), spec(1), spec(2), spec(3)] * 2
    oshape = jax.ShapeDtypeStruct((nsteps * _C, 128), jnp.float32)
    return pl.pallas_call(
        _relayout_body,
        grid=(nsteps,),
        in_specs=specs,
        out_specs=[pl.BlockSpec((_C, 128), lambda i: (i, 0))] * 2,
        out_shape=[oshape, oshape],
        compiler_params=pltpu.CompilerParams(
            dimension_semantics=("parallel",)),
    )(pos_t, pos_t, pos_t, pos_t, meta_t, meta_t, meta_t, meta_t)


def _perm_big(idx):
    b = idx // _C
    return ((b // 4) * _C + idx % _C) * 4 + b % 4


def _relayout_type_body(tin, o):
    q = TYPE_ROWS // 4
    for k in range(4):
        o[:, 32 * k:32 * k + 32] = jnp.transpose(tin[:, k * q:(k + 1) * q])


def _tc_relayout_type(table_t):
    q = TYPE_ROWS // 4
    return pl.pallas_call(
        _relayout_type_body,
        grid=(1,),
        in_specs=[pl.BlockSpec((32, TYPE_ROWS), lambda i: (0, 0))],
        out_specs=pl.BlockSpec((q, 128), lambda i: (0, 0)),
        out_shape=jax.ShapeDtypeStruct((q, 128), jnp.float32),
    )(table_t)


def _sc_gather3(t1, t2, t3, i1, i2, i3):
    mesh = plsc.VectorSubcoreMesh(core_axis_name="c", subcore_axis_name="s")

    @functools.partial(
        pl.kernel,
        mesh=mesh,
        out_type=[jax.ShapeDtypeStruct((N, D), jnp.float32)] * 3,
        scratch_types=[
            pltpu.VMEM((_BPW,), jnp.int32),
            pltpu.VMEM((_BPW,), jnp.int32),
            pltpu.VMEM((_BPW,), jnp.int32),
            pltpu.VMEM((_BPW, D), jnp.float32),
            pltpu.VMEM((_BPW, D), jnp.float32),
            pltpu.VMEM((_BPW, D), jnp.float32),
            pltpu.SemaphoreType.DMA,
            pltpu.SemaphoreType.DMA,
            pltpu.SemaphoreType.DMA,
        ],
        compiler_params=pltpu.CompilerParams(use_tc_tiling_on_sc=False),
    )
    def k(ta, tb, tc, ia_, ib_, ic_, o_a, o_b, o_c,
          iv_a, iv_b, iv_c, r_a, r_b, r_c, s_a, s_b, s_c):
        wid = lax.axis_index("s") * _NC + lax.axis_index("c")
        base = wid * _BPW
        pltpu.sync_copy(ia_.at[pl.ds(base, _BPW)], iv_a)
        pltpu.sync_copy(ib_.at[pl.ds(base, _BPW)], iv_b)
        pltpu.sync_copy(ic_.at[pl.ds(base, _BPW)], iv_c)
        c1 = pltpu.async_copy(ta.at[iv_a], r_a, s_a)
        c2 = pltpu.async_copy(tb.at[iv_b], r_b, s_b)
        c3 = pltpu.async_copy(tc.at[iv_c], r_c, s_c)
        c1.wait()
        c2.wait()
        c3.wait()
        pltpu.sync_copy(r_a, o_a.at[pl.ds(base, _BPW)])
        pltpu.sync_copy(r_b, o_b.at[pl.ds(base, _BPW)])
        pltpu.sync_copy(r_c, o_c.at[pl.ds(base, _BPW)])

    return k(t1, t2, t3, i1, i2, i3)


def _gelu(x):
    return x * 0.5 * (1.0 + lax.erf(x * 0.7071067811865476))


def _ln(x, g, b):
    m = jnp.mean(x, axis=-1, keepdims=True)
    d = x - m
    v = jnp.mean(d * d, axis=-1, keepdims=True)
    return d * lax.rsqrt(v + 1e-5) * g + b


def _avg_mat(w):
    return jnp.full((w, w), 1.0 / w, dtype=jnp.float32)


def _ln_mxu(x, g, b):
    mm = _avg_mat(x.shape[-1])
    m = jnp.dot(x, mm, preferred_element_type=jnp.float32)
    d = x - m
    v = jnp.dot(d * d, mm, preferred_element_type=jnp.float32)
    return d * lax.rsqrt(v + 1e-5) * g + b


def _dense_body(te, pe, me, db,
                dbW, dbb, dbg, dbbeta,
                tmW, tmb, tmg, tmbeta,
                tfW, tfb, tfg, tfbeta,
                fW1, fW2, fW3, fW4, fb, fg, fbeta, o):
    f32 = jnp.float32
    b4 = te.shape[0]
    tec = jnp.concatenate([te[:, 32 * a:32 * a + 32] for a in range(4)], 0)
    pec = jnp.concatenate([pe[:, 32 * a:32 * a + 32] for a in range(4)], 0)
    mec = jnp.concatenate([me[:, 32 * a:32 * a + 32] for a in range(4)], 0)
    dbc = jnp.concatenate([db[:, 2 * a:2 * a + 2] for a in range(4)], 0)
    db_h = jnp.dot(dbc, dbW[...], preferred_element_type=f32) + dbb[...]
    db_e = _gelu(_ln_mxu(db_h, dbg[...], dbbeta[...]))
    t = jnp.dot(mec, tmW[...], preferred_element_type=f32) + tmb[...]
    t = _gelu(_ln_mxu(t, tmg[...], tmbeta[...]))
    t = jnp.dot(t, tfW[...], preferred_element_type=f32) + tfb[...]
    t = _gelu(_ln_mxu(t, tfg[...], tfbeta[...]))
    acc = (jnp.dot(tec, fW1[...], preferred_element_type=f32)
           + jnp.dot(pec, fW2[...], preferred_element_type=f32)
           + jnp.dot(db_e, fW3[...], preferred_element_type=f32)
           + jnp.dot(t, fW4[...], preferred_element_type=f32)
           + fb[...])
    res = _gelu(_ln_mxu(acc, fg[...], fbeta[...]))
    packed = jnp.stack([res[a * b4:(a + 1) * b4] for a in range(4)], 1)
    o[...] = packed.reshape(o.shape)


def _tc_dense(te4, pe4, me4, db4,
              db_W, db_b, db_g, db_beta,
              tm_W, tm_b, tm_g, tm_beta,
              tfin_W, tfin_b, tfin_g, tfin_beta,
              fin_W, fin_b, fin_g, fin_beta,
              block=4096):
    grid = (N // block,)
    b4 = block // 4

    def row(d):
        return pl.BlockSpec((b4, d), lambda i: (i, 0))

    def full(a):
        return pl.BlockSpec(a.shape, lambda i: (0,) * a.ndim)

    fW1 = fin_W[0:32]
    fW2 = fin_W[32:64]
    fW3 = fin_W[64:96]
    fW4 = fin_W[96:160]
    vecs = [db_b, db_g, db_beta, tm_b, tm_g, tm_beta,
            tfin_b, tfin_g, tfin_beta, fin_b, fin_g, fin_beta]
    (db_b, db_g, db_beta, tm_b, tm_g, tm_beta,
     tfin_b, tfin_g, tfin_beta, fin_b, fin_g, fin_beta) = [
        v.reshape(1, -1) for v in vecs]

    args = (te4, pe4, me4, db4,
            db_W, db_b, db_g, db_beta,
            tm_W, tm_b, tm_g, tm_beta,
            tfin_W, tfin_b, tfin_g, tfin_beta,
            fW1, fW2, fW3, fW4, fin_b, fin_g, fin_beta)
    specs = [row(128), row(128), row(128), row(8)] + [full(a) for a in args[4:]]

    return pl.pallas_call(
        _dense_body,
        grid=grid,
        in_specs=specs,
        out_specs=pl.BlockSpec((block, 128), lambda i: (i, 0)),
        out_shape=jax.ShapeDtypeStruct((N, 128), jnp.float32),
        compiler_params=pltpu.CompilerParams(
            dimension_semantics=("parallel",)),
    )(*args)


def kernel(features, type_table, pos_table, table_meta,
           db_W, db_b, db_g, db_beta,
           tm_W, tm_b, tm_g, tm_beta,
           tfin_W, tfin_b, tfin_g, tfin_beta,
           fin_W, fin_b, fin_g, fin_beta):
    idx_type = features[:, 0].astype(jnp.int32)
    idx_pos = features[:, 1].astype(jnp.int32)
    idx_tab = features[:, 6].astype(jnp.int32)
    db4 = (features.T[2:4].reshape(2, N // 4, 4)
           .transpose(1, 2, 0).reshape(N // 4, 8))
    tt_lin = _tc_relayout_type(type_table.T).reshape(TYPE_ROWS, D)
    pt_lin, tm_lin = _tc_relayout_big(pos_table.T, table_meta.T)
    pt_lin = pt_lin.reshape(pt_lin.shape[0] * 4, D)
    tm_lin = tm_lin.reshape(tm_lin.shape[0] * 4, D)
    idx_type = (idx_type % (TYPE_ROWS // 4)) * 4 + idx_type // (TYPE_ROWS // 4)
    idx_pos = _perm_big(idx_pos)
    idx_tab = _perm_big(idx_tab)
    te, pe, me = _sc_gather3(tt_lin, pt_lin, tm_lin,
                             idx_type, idx_pos, idx_tab)
    te4 = te.reshape(N // 4, 128)
    pe4 = pe.reshape(N // 4, 128)
    me4 = me.reshape(N // 4, 128)
    return _tc_dense(te4, pe4, me4, db4,
                     db_W, db_b, db_g, db_beta,
                     tm_W, tm_b, tm_g, tm_beta,
                     tfin_W, tfin_b, tfin_g, tfin_beta,
                     fin_W, fin_b, fin_g, fin_beta)


# speedup vs baseline: 1.1159x; 1.0181x over previous
import functools

import jax
import jax.numpy as jnp
from jax import lax
from jax.experimental import pallas as pl
from jax.experimental.pallas import tpu as pltpu
from jax.experimental.pallas import tpu_sc as plsc

N = 16384
D = 32
TYPE_ROWS = 1000
BIG_ROWS = 100000
_NC = 2
_NS = 16
_NW = _NC * _NS
_BPW = N // _NW

_C = 2048


def _eye_off(k):
    r = lax.broadcasted_iota(jnp.int32, (32, 128), 0)
    c = lax.broadcasted_iota(jnp.int32, (32, 128), 1)
    return jnp.where(c == r + 32 * k, 1.0, 0.0).astype(jnp.float32)


def _mxu_t4(xs):
    acc = None
    for k, x in enumerate(xs):
        p = lax.dot_general(x, _eye_off(k), (((0,), (0,)), ((), ())),
                            preferred_element_type=jnp.float32)
        acc = p if acc is None else acc + p
    return acc


def _relayout_body(p0, p1, p2, p3, m0, m1, m2, m3, op, om):
    i = pl.program_id(0)
    col = lax.broadcasted_iota(jnp.int32, (32, _C), 1)
    ps, ms = [p0, p1, p2, p3], [m0, m1, m2, m3]
    ok = [(4 * i + k) * _C + col < BIG_ROWS for k in range(4)]
    op[...] = _mxu_t4([jnp.where(ok[k], ps[k][...], 0.0) for k in range(4)])
    om[...] = _mxu_t4([jnp.where(ok[k], ms[k][...], 0.0) for k in range(4)])


def _tc_relayout_big(pos_t, meta_t):
    v = pos_t.shape[1]
    g = -(-v // _C)
    nsteps = -(-g // 4)

    def spec(k):
        return pl.BlockSpec(
            (32, _C), lambda i, k=k: (0, jnp.minimum(4 * i + k, g - 1)))

    specs = [spec(0), spec(1), spec(2), spec(3)] * 2
    oshape = jax.ShapeDtypeStruct((nsteps * _C, 128), jnp.float32)
    return pl.pallas_call(
        _relayout_body,
        grid=(nsteps,),
        in_specs=specs,
        out_specs=[pl.BlockSpec((_C, 128), lambda i: (i, 0))] * 2,
        out_shape=[oshape, oshape],
        compiler_params=pltpu.CompilerParams(
            dimension_semantics=("parallel",)),
    )(pos_t, pos_t, pos_t, pos_t, meta_t, meta_t, meta_t, meta_t)


def _perm_big(idx):
    b = idx // _C
    return ((b // 4) * _C + idx % _C) * 4 + b % 4


def _relayout_type_body(tin, o):
    q = TYPE_ROWS // 4
    for k in range(4):
        o[:, 32 * k:32 * k + 32] = jnp.transpose(tin[:, k * q:(k + 1) * q])


def _tc_relayout_type(table_t):
    q = TYPE_ROWS // 4
    return pl.pallas_call(
        _relayout_type_body,
        grid=(1,),
        in_specs=[pl.BlockSpec((32, TYPE_ROWS), lambda i: (0, 0))],
        out_specs=pl.BlockSpec((q, 128), lambda i: (0, 0)),
        out_shape=jax.ShapeDtypeStruct((q, 128), jnp.float32),
    )(table_t)


def _sc_gather3(t1, t2, t3, i1, i2, i3):
    mesh = plsc.VectorSubcoreMesh(core_axis_name="c", subcore_axis_name="s")

    @functools.partial(
        pl.kernel,
        mesh=mesh,
        out_type=[jax.ShapeDtypeStruct((N, D), jnp.float32)] * 3,
        scratch_types=[
            pltpu.VMEM((_BPW,), jnp.int32),
            pltpu.VMEM((_BPW,), jnp.int32),
            pltpu.VMEM((_BPW,), jnp.int32),
            pltpu.VMEM((_BPW, D), jnp.float32),
            pltpu.VMEM((_BPW, D), jnp.float32),
            pltpu.VMEM((_BPW, D), jnp.float32),
            pltpu.SemaphoreType.DMA,
            pltpu.SemaphoreType.DMA,
            pltpu.SemaphoreType.DMA,
        ],
        compiler_params=pltpu.CompilerParams(use_tc_tiling_on_sc=False),
    )
    def k(ta, tb, tc, ia_, ib_, ic_, o_a, o_b, o_c,
          iv_a, iv_b, iv_c, r_a, r_b, r_c, s_a, s_b, s_c):
        wid = lax.axis_index("s") * _NC + lax.axis_index("c")
        base = wid * _BPW
        pltpu.sync_copy(ia_.at[pl.ds(base, _BPW)], iv_a)
        pltpu.sync_copy(ib_.at[pl.ds(base, _BPW)], iv_b)
        pltpu.sync_copy(ic_.at[pl.ds(base, _BPW)], iv_c)
        c1 = pltpu.async_copy(ta.at[iv_a], r_a, s_a)
        c2 = pltpu.async_copy(tb.at[iv_b], r_b, s_b)
        c3 = pltpu.async_copy(tc.at[iv_c], r_c, s_c)
        c1.wait()
        c2.wait()
        c3.wait()
        pltpu.sync_copy(r_a, o_a.at[pl.ds(base, _BPW)])
        pltpu.sync_copy(r_b, o_b.at[pl.ds(base, _BPW)])
        pltpu.sync_copy(r_c, o_c.at[pl.ds(base, _BPW)])

    return k(t1, t2, t3, i1, i2, i3)


def _gelu(x):
    return x * 0.5 * (1.0 + lax.erf(x * 0.7071067811865476))


def _ln(x, g, b):
    m = jnp.mean(x, axis=-1, keepdims=True)
    d = x - m
    v = jnp.mean(d * d, axis=-1, keepdims=True)
    return d * lax.rsqrt(v + 1e-5) * g + b


def _avg_mat(w):
    return jnp.full((w, w), 1.0 / w, dtype=jnp.float32)


def _ln_mxu(x, g, b):
    mm = _avg_mat(x.shape[-1])
    m = jnp.dot(x, mm, preferred_element_type=jnp.float32)
    d = x - m
    v = jnp.dot(d * d, mm, preferred_element_type=jnp.float32)
    return d * lax.rsqrt(v + 1e-5) * g + b


def _dense_body(te, pe, me, db,
                dbW, dbb, dbg, dbbeta,
                tmW, tmb, tmg, tmbeta,
                tfW, tfb, tfg, tfbeta,
                fW1, fW2, fW3, fW4, fb, fg, fbeta, o):
    f32 = jnp.float32
    b4 = te.shape[0]
    tec = jnp.concatenate([te[:, 32 * a:32 * a + 32] for a in range(4)], 0)
    pec = jnp.concatenate([pe[:, 32 * a:32 * a + 32] for a in range(4)], 0)
    mec = jnp.concatenate([me[:, 32 * a:32 * a + 32] for a in range(4)], 0)
    dbc = jnp.concatenate([db[:, 2 * a:2 * a + 2] for a in range(4)], 0)
    db_h = jnp.dot(dbc, dbW[...], preferred_element_type=f32) + dbb[...]
    db_e = _gelu(_ln_mxu(db_h, dbg[...], dbbeta[...]))
    t = jnp.dot(mec, tmW[...], preferred_element_type=f32) + tmb[...]
    t = _gelu(_ln_mxu(t, tmg[...], tmbeta[...]))
    t = jnp.dot(t, tfW[...], preferred_element_type=f32) + tfb[...]
    t = _gelu(_ln_mxu(t, tfg[...], tfbeta[...]))
    acc = (jnp.dot(tec, fW1[...], preferred_element_type=f32)
           + jnp.dot(pec, fW2[...], preferred_element_type=f32)
           + jnp.dot(db_e, fW3[...], preferred_element_type=f32)
           + jnp.dot(t, fW4[...], preferred_element_type=f32)
           + fb[...])
    res = _gelu(_ln_mxu(acc, fg[...], fbeta[...]))
    packed = jnp.stack([res[a * b4:(a + 1) * b4] for a in range(4)], 1)
    o[...] = packed.reshape(o.shape)


def _tc_dense(te4, pe4, me4, db4,
              db_W, db_b, db_g, db_beta,
              tm_W, tm_b, tm_g, tm_beta,
              tfin_W, tfin_b, tfin_g, tfin_beta,
              fin_W, fin_b, fin_g, fin_beta,
              block=4096):
    grid = (N // block,)
    b4 = block // 4

    def row(d):
        return pl.BlockSpec((b4, d), lambda i: (i, 0))

    def full(a):
        return pl.BlockSpec(a.shape, lambda i: (0,) * a.ndim)

    fW1 = fin_W[0:32]
    fW2 = fin_W[32:64]
    fW3 = fin_W[64:96]
    fW4 = fin_W[96:160]
    vecs = [db_b, db_g, db_beta, tm_b, tm_g, tm_beta,
            tfin_b, tfin_g, tfin_beta, fin_b, fin_g, fin_beta]
    (db_b, db_g, db_beta, tm_b, tm_g, tm_beta,
     tfin_b, tfin_g, tfin_beta, fin_b, fin_g, fin_beta) = [
        v.reshape(1, -1) for v in vecs]

    args = (te4, pe4, me4, db4,
            db_W, db_b, db_g, db_beta,
            tm_W, tm_b, tm_g, tm_beta,
            tfin_W, tfin_b, tfin_g, tfin_beta,
            fW1, fW2, fW3, fW4, fin_b, fin_g, fin_beta)
    specs = [row(128), row(128), row(128), row(8)] + [full(a) for a in args[4:]]

    return pl.pallas_call(
        _dense_body,
        grid=grid,
        in_specs=specs,
        out_specs=pl.BlockSpec((block, 128), lambda i: (i, 0)),
        out_shape=jax.ShapeDtypeStruct((N, 128), jnp.float32),
        compiler_params=pltpu.CompilerParams(
            dimension_semantics=("parallel",)),
    )(*args)


def kernel(features, type_table, pos_table, table_meta,
           db_W, db_b, db_g, db_beta,
           tm_W, tm_b, tm_g, tm_beta,
           tfin_W, tfin_b, tfin_g, tfin_beta,
           fin_W, fin_b, fin_g, fin_beta):
    ids = features.T[jnp.array([0, 1, 6])].astype(jnp.int32)
    idx_type, idx_pos, idx_tab = ids[0], ids[1], ids[2]
    db4 = (features.T[2:4].reshape(2, N // 4, 4)
           .transpose(1, 2, 0).reshape(N // 4, 8))
    tt_lin = _tc_relayout_type(type_table.T).reshape(TYPE_ROWS, D)
    pt_lin, tm_lin = _tc_relayout_big(pos_table.T, table_meta.T)
    pt_lin = pt_lin.reshape(pt_lin.shape[0] * 4, D)
    tm_lin = tm_lin.reshape(tm_lin.shape[0] * 4, D)
    idx_type = (idx_type % (TYPE_ROWS // 4)) * 4 + idx_type // (TYPE_ROWS // 4)
    idx_pos = _perm_big(idx_pos)
    idx_tab = _perm_big(idx_tab)
    te, pe, me = _sc_gather3(tt_lin, pt_lin, tm_lin,
                             idx_type, idx_pos, idx_tab)
    te4 = te.reshape(N // 4, 128)
    pe4 = pe.reshape(N // 4, 128)
    me4 = me.reshape(N // 4, 128)
    return _tc_dense(te4, pe4, me4, db4,
                     db_W, db_b, db_g, db_beta,
                     tm_W, tm_b, tm_g, tm_beta,
                     tfin_W, tfin_b, tfin_g, tfin_beta,
                     fin_W, fin_b, fin_g, fin_beta)
